# Initial kernel scaffold; baseline (speedup 1.0000x reference)
import functools

import jax
import jax.numpy as jnp
from jax import lax
from jax.experimental import pallas as pl
from jax.experimental.pallas import tpu as pltpu
from jax.experimental.pallas import tpu_sc as plsc

B, L = 4096, 128
VOCAB, DIM = 100000, 64

NC, NS, LANES = 2, 16, 16
NW = NC * NS
SEQ_PER_W = B // NW
NV = DIM // LANES

_mesh = plsc.VectorSubcoreMesh(core_axis_name="c", subcore_axis_name="s")


@functools.partial(
    pl.kernel,
    mesh=_mesh,
    out_type=jax.ShapeDtypeStruct((B, DIM), jnp.float32),
    scratch_types=[
        pltpu.VMEM((SEQ_PER_W, L), jnp.int32),
        pltpu.VMEM((L, DIM), jnp.float32),
        pltpu.VMEM((1, DIM), jnp.float32),
        pltpu.VMEM((SEQ_PER_W, DIM), jnp.float32),
        pltpu.SemaphoreType.DMA,
    ],
)
def _embed_pool(ids_hbm, table_hbm, out_hbm, ids_v, rows_v, t0_v, out_v, sem):
    wid = lax.axis_index("s") * NC + lax.axis_index("c")
    base = wid * SEQ_PER_W

    pltpu.sync_copy(ids_hbm.at[pl.ds(base, SEQ_PER_W)], ids_v)
    pltpu.sync_copy(table_hbm.at[pl.ds(0, 1)], t0_v)

    def seq_body(j, carry):
        pltpu.async_copy(table_hbm.at[ids_v.at[j]], rows_v, sem).wait()

        def row_body(r, accs):
            a0, a1, a2, a3 = accs
            a0 = a0 + rows_v[r, pl.ds(0, LANES)]
            a1 = a1 + rows_v[r, pl.ds(LANES, LANES)]
            a2 = a2 + rows_v[r, pl.ds(2 * LANES, LANES)]
            a3 = a3 + rows_v[r, pl.ds(3 * LANES, LANES)]
            return (a0, a1, a2, a3)

        zero = jnp.zeros((LANES,), jnp.float32)
        accs = lax.fori_loop(0, L, row_body, (zero, zero, zero, zero))

        n0 = jnp.zeros((LANES,), jnp.int32)
        for k in range(L // LANES):
            n0 = n0 + plsc.all_reduce_population_count(
                ids_v[j, pl.ds(k * LANES, LANES)] == 0)
        n0f = n0.astype(jnp.float32)
        cnt = jnp.float32(L) - n0f

        for d in range(NV):
            t0 = t0_v[0, pl.ds(d * LANES, LANES)]
            out_v[j, pl.ds(d * LANES, LANES)] = (accs[d] - n0f * t0) / cnt
        return carry

    lax.fori_loop(0, SEQ_PER_W, seq_body, 0)
    pltpu.sync_copy(out_v, out_hbm.at[pl.ds(base, SEQ_PER_W)])


def kernel(input_ids, table):
    ids = jnp.asarray(input_ids, jnp.int32)
    return _embed_pool(ids, table)


# speedup vs baseline: 7.0860x; 7.0860x over previous
import functools

import jax
import jax.numpy as jnp
from jax import lax
from jax.experimental import pallas as pl
from jax.experimental.pallas import tpu as pltpu
from jax.experimental.pallas import tpu_sc as plsc

B, L = 4096, 128
VOCAB, DIM = 100000, 64

NC, NS, LANES = 2, 16, 16
NW = NC * NS
SEQ_PER_W = B // NW
NV = DIM // LANES

_mesh = plsc.VectorSubcoreMesh(core_axis_name="c", subcore_axis_name="s")


@functools.partial(
    pl.kernel,
    mesh=_mesh,
    out_type=jax.ShapeDtypeStruct((B, DIM), jnp.float32),
    scratch_types=[
        pltpu.VMEM((SEQ_PER_W, L), jnp.int32),
        pltpu.VMEM((L, DIM), jnp.float32),
        pltpu.VMEM((1, DIM), jnp.float32),
        pltpu.VMEM((SEQ_PER_W, DIM), jnp.float32),
        pltpu.SemaphoreType.DMA,
    ],
    compiler_params=pltpu.CompilerParams(use_tc_tiling_on_sc=False),
)
def _embed_pool(ids_hbm, table_hbm, out_hbm, ids_v, rows_v, t0_v, out_v, sem):
    wid = lax.axis_index("s") * NC + lax.axis_index("c")
    base = wid * SEQ_PER_W

    pltpu.sync_copy(ids_hbm.at[pl.ds(base, SEQ_PER_W)], ids_v)
    pltpu.sync_copy(table_hbm.at[pl.ds(0, 1)], t0_v)

    def seq_body(j, carry):
        pltpu.async_copy(table_hbm.at[ids_v.at[j]], rows_v, sem).wait()

        def row_body(r, accs):
            a0, a1, a2, a3 = accs
            a0 = a0 + rows_v[r, pl.ds(0, LANES)]
            a1 = a1 + rows_v[r, pl.ds(LANES, LANES)]
            a2 = a2 + rows_v[r, pl.ds(2 * LANES, LANES)]
            a3 = a3 + rows_v[r, pl.ds(3 * LANES, LANES)]
            return (a0, a1, a2, a3)

        zero = jnp.zeros((LANES,), jnp.float32)
        accs = lax.fori_loop(0, L, row_body, (zero, zero, zero, zero))

        n0v = jnp.zeros((LANES,), jnp.int32)
        for k in range(L // LANES):
            n0v = n0v + jnp.where(ids_v[j, pl.ds(k * LANES, LANES)] == 0, 1, 0)
        lane = lax.iota(jnp.int32, LANES)
        dnums = lax.GatherDimensionNumbers(
            offset_dims=(), collapsed_slice_dims=(0,), start_index_map=(0,))
        for sh in (1, 2, 4, 8):
            perm = (lane ^ sh)[:, None]
            n0v = n0v + lax.gather(
                n0v, perm, dnums, (1,),
                mode=lax.GatherScatterMode.PROMISE_IN_BOUNDS)
        n0f = n0v.astype(jnp.float32)
        cnt = jnp.float32(L) - n0f

        for d in range(NV):
            t0 = t0_v[0, pl.ds(d * LANES, LANES)]
            out_v[j, pl.ds(d * LANES, LANES)] = (accs[d] - n0f * t0) / cnt
        return carry

    lax.fori_loop(0, SEQ_PER_W, seq_body, 0)
    pltpu.sync_copy(out_v, out_hbm.at[pl.ds(base, SEQ_PER_W)])


def kernel(input_ids, table):
    ids = jnp.asarray(input_ids, jnp.int32)
    return _embed_pool(ids, table)


# speedup vs baseline: 13.9950x; 1.9750x over previous
import functools

import jax
import jax.numpy as jnp
from jax import lax
from jax.experimental import pallas as pl
from jax.experimental.pallas import tpu as pltpu
from jax.experimental.pallas import tpu_sc as plsc

B, L = 4096, 128
VOCAB, DIM = 100000, 64

NC, NS, LANES = 2, 16, 16
NW = NC * NS
SEQ_PER_W = B // NW
NV = DIM // LANES
NBUF = 4
UNROLL = 4

_mesh = plsc.VectorSubcoreMesh(core_axis_name="c", subcore_axis_name="s")


@functools.partial(
    pl.kernel,
    mesh=_mesh,
    out_type=jax.ShapeDtypeStruct((B, DIM), jnp.float32),
    scratch_types=[
        pltpu.VMEM((SEQ_PER_W, L), jnp.int32),
        pltpu.VMEM((NBUF, L, DIM), jnp.float32),
        pltpu.VMEM((1, DIM), jnp.float32),
        pltpu.VMEM((SEQ_PER_W, DIM), jnp.float32),
        pltpu.SemaphoreType.DMA,
        pltpu.SemaphoreType.DMA,
        pltpu.SemaphoreType.DMA,
        pltpu.SemaphoreType.DMA,
    ],
    compiler_params=pltpu.CompilerParams(use_tc_tiling_on_sc=False),
)
def _embed_pool(ids_hbm, table_hbm, out_hbm, ids_v, rows_v, t0_v, out_v,
                sem0, sem1, sem2, sem3):
    sems = (sem0, sem1, sem2, sem3)
    wid = lax.axis_index("s") * NC + lax.axis_index("c")
    base = wid * SEQ_PER_W

    pltpu.sync_copy(ids_hbm.at[pl.ds(base, SEQ_PER_W)], ids_v)
    pltpu.sync_copy(table_hbm.at[pl.ds(0, 1)], t0_v)

    def start(s, b):
        pltpu.async_copy(table_hbm.at[ids_v.at[s]], rows_v.at[b], sems[b])

    def wait(s, b):
        pltpu.make_async_copy(
            table_hbm.at[ids_v.at[s]], rows_v.at[b], sems[b]).wait()

    def process(s, b):
        rv = rows_v.at[b]

        def row_body(r, accs):
            new = []
            for u in range(UNROLL):
                au = list(accs[u * NV:(u + 1) * NV])
                for d in range(NV):
                    au[d] = au[d] + rv[r * UNROLL + u, pl.ds(d * LANES, LANES)]
                new.extend(au)
            return tuple(new)

        zero = jnp.zeros((LANES,), jnp.float32)
        accs = lax.fori_loop(0, L // UNROLL, row_body, (zero,) * (UNROLL * NV))

        n0v = jnp.zeros((LANES,), jnp.int32)
        for k in range(L // LANES):
            n0v = n0v + jnp.where(ids_v[s, pl.ds(k * LANES, LANES)] == 0, 1, 0)
        lane = lax.iota(jnp.int32, LANES)
        dnums = lax.GatherDimensionNumbers(
            offset_dims=(), collapsed_slice_dims=(0,), start_index_map=(0,))
        for sh in (1, 2, 4, 8):
            perm = (lane ^ sh)[:, None]
            n0v = n0v + lax.gather(
                n0v, perm, dnums, (1,),
                mode=lax.GatherScatterMode.PROMISE_IN_BOUNDS)
        n0f = n0v.astype(jnp.float32)
        cnt = jnp.float32(L) - n0f

        for d in range(NV):
            tot = accs[d]
            for u in range(1, UNROLL):
                tot = tot + accs[u * NV + d]
            t0 = t0_v[0, pl.ds(d * LANES, LANES)]
            out_v[s, pl.ds(d * LANES, LANES)] = (tot - n0f * t0) / cnt

    for b in range(NBUF):
        start(b, b)

    def group_body(g, carry):
        for b in range(NBUF):
            s = g * NBUF + b
            wait(s, b)
            process(s, b)

            @pl.when(s + NBUF < SEQ_PER_W)
            def _():
                start(s + NBUF, b)
        return carry

    lax.fori_loop(0, SEQ_PER_W // NBUF, group_body, 0)
    pltpu.sync_copy(out_v, out_hbm.at[pl.ds(base, SEQ_PER_W)])


def kernel(input_ids, table):
    ids = jnp.asarray(input_ids, jnp.int32)
    return _embed_pool(ids, table)


# speedup vs baseline: 14.5031x; 1.0363x over previous
import functools

import jax
import jax.numpy as jnp
from jax import lax
from jax.experimental import pallas as pl
from jax.experimental.pallas import tpu as pltpu
from jax.experimental.pallas import tpu_sc as plsc

B, L = 4096, 128
VOCAB, DIM = 100000, 64

NC, NS, LANES = 2, 16, 16
NW = NC * NS
SEQ_PER_W = B // NW
NV = DIM // LANES
NBUF = 8
UNROLL = 8

_mesh = plsc.VectorSubcoreMesh(core_axis_name="c", subcore_axis_name="s")


@functools.partial(
    pl.kernel,
    mesh=_mesh,
    out_type=jax.ShapeDtypeStruct((B, DIM), jnp.float32),
    scratch_types=[
        pltpu.VMEM((SEQ_PER_W, L), jnp.int32),
        pltpu.VMEM((NBUF, L, DIM), jnp.float32),
        pltpu.VMEM((1, DIM), jnp.float32),
        pltpu.VMEM((SEQ_PER_W, DIM), jnp.float32),
    ] + [pltpu.SemaphoreType.DMA] * NBUF,
    compiler_params=pltpu.CompilerParams(use_tc_tiling_on_sc=False),
)
def _embed_pool(ids_hbm, table_hbm, out_hbm, ids_v, rows_v, t0_v, out_v,
                *sems):
    wid = lax.axis_index("s") * NC + lax.axis_index("c")
    base = wid * SEQ_PER_W

    pltpu.sync_copy(ids_hbm.at[pl.ds(base, SEQ_PER_W)], ids_v)
    pltpu.sync_copy(table_hbm.at[pl.ds(0, 1)], t0_v)

    def start(s, b):
        pltpu.async_copy(table_hbm.at[ids_v.at[s]], rows_v.at[b], sems[b])

    def wait(s, b):
        pltpu.make_async_copy(
            table_hbm.at[ids_v.at[s]], rows_v.at[b], sems[b]).wait()

    def process(s, b):
        rv = rows_v.at[b]

        def row_body(r, accs):
            new = []
            for u in range(UNROLL):
                au = list(accs[u * NV:(u + 1) * NV])
                for d in range(NV):
                    au[d] = au[d] + rv[r * UNROLL + u, pl.ds(d * LANES, LANES)]
                new.extend(au)
            return tuple(new)

        zero = jnp.zeros((LANES,), jnp.float32)
        accs = lax.fori_loop(0, L // UNROLL, row_body, (zero,) * (UNROLL * NV))

        n0v = jnp.zeros((LANES,), jnp.int32)
        for k in range(L // LANES):
            n0v = n0v + jnp.where(ids_v[s, pl.ds(k * LANES, LANES)] == 0, 1, 0)
        lane = lax.iota(jnp.int32, LANES)
        dnums = lax.GatherDimensionNumbers(
            offset_dims=(), collapsed_slice_dims=(0,), start_index_map=(0,))
        for sh in (1, 2, 4, 8):
            perm = (lane ^ sh)[:, None]
            n0v = n0v + lax.gather(
                n0v, perm, dnums, (1,),
                mode=lax.GatherScatterMode.PROMISE_IN_BOUNDS)
        n0f = n0v.astype(jnp.float32)
        cnt = jnp.float32(L) - n0f

        for d in range(NV):
            tot = accs[d]
            for u in range(1, UNROLL):
                tot = tot + accs[u * NV + d]
            t0 = t0_v[0, pl.ds(d * LANES, LANES)]
            out_v[s, pl.ds(d * LANES, LANES)] = (tot - n0f * t0) / cnt

    for b in range(NBUF):
        start(b, b)

    def group_body(g, carry):
        for b in range(NBUF):
            s = g * NBUF + b
            wait(s, b)
            process(s, b)

            @pl.when(s + NBUF < SEQ_PER_W)
            def _():
                start(s + NBUF, b)
        return carry

    lax.fori_loop(0, SEQ_PER_W // NBUF, group_body, 0)
    pltpu.sync_copy(out_v, out_hbm.at[pl.ds(base, SEQ_PER_W)])


def kernel(input_ids, table):
    ids = jnp.asarray(input_ids, jnp.int32)
    return _embed_pool(ids, table)
